# baseline (speedup 1.0000x reference)
import functools

import jax
import jax.numpy as jnp
from jax import lax
from jax.experimental import pallas as pl
from jax.experimental.pallas import tpu as pltpu
from jax.experimental.pallas import tpu_sc as plsc

TOKEN_BLOCK = 2048
LANES = 16
NWORKERS = 32


def _proj_softmax_kernel(x_ref, w_ref, logits_ref, aff_ref):
    x = x_ref[...]
    w = w_ref[...]
    logits = jax.lax.dot_general(
        x, w, (((1,), (1,)), ((), ())), preferred_element_type=jnp.float32
    )
    logits_ref[...] = logits
    m = jnp.max(logits, axis=1, keepdims=True)
    e = jnp.exp(logits - m)
    s = jnp.sum(e, axis=1, keepdims=True)
    aff_ref[...] = e / s


def _tc_proj_softmax(x, W, T, H, E):
    tb = TOKEN_BLOCK
    return pl.pallas_call(
        _proj_softmax_kernel,
        grid=(T // tb,),
        in_specs=[
            pl.BlockSpec((tb, H), lambda i: (i, 0)),
            pl.BlockSpec((E, H), lambda i: (0, 0)),
        ],
        out_specs=[
            pl.BlockSpec((tb, E), lambda i: (i, 0)),
            pl.BlockSpec((tb, E), lambda i: (i, 0)),
        ],
        out_shape=[
            jax.ShapeDtypeStruct((T, E), jnp.float32),
            jax.ShapeDtypeStruct((T, E), jnp.float32),
        ],
        compiler_params=pltpu.CompilerParams(
            dimension_semantics=("parallel",),
        ),
    )(x, W)


def _sc_top2_body(aff_hbm, out_hbm, chunk, outc, *, tpw, n_experts):
    nvr = n_experts // LANES
    wid = lax.axis_index("s") * 2 + lax.axis_index("c")
    base = wid * tpw
    pltpu.sync_copy(aff_hbm.at[pl.ds(base * n_experts, tpw * n_experts)], chunk)

    lane = lax.iota(jnp.int32, LANES)
    gid = [lane + LANES * k for k in range(nvr)]
    big = jnp.full((LANES,), n_experts, jnp.int32)
    ninf = jnp.full((LANES,), -jnp.inf, jnp.float32)

    def group_body(g, _):
        i1acc = jnp.zeros((LANES,), jnp.int32)
        i2acc = jnp.zeros((LANES,), jnp.int32)
        for j in range(LANES):
            t = g * LANES + j
            vs = [chunk[pl.ds(t * n_experts + LANES * k, LANES)]
                  for k in range(nvr)]
            m = vs[0]
            for k in range(1, nvr):
                m = jnp.maximum(m, vs[k])
            m1 = jnp.max(m, axis=0)
            m1v = jnp.full((LANES,), m1, jnp.float32)
            cmin = big
            for k in range(nvr):
                cmin = jnp.minimum(cmin, jnp.where(vs[k] == m1v, gid[k], big))
            i1 = jnp.min(cmin, axis=0)
            i1v = jnp.full((LANES,), i1, jnp.int32)
            vs2 = [jnp.where(gid[k] == i1v, ninf, vs[k]) for k in range(nvr)]
            m2 = vs2[0]
            for k in range(1, nvr):
                m2 = jnp.maximum(m2, vs2[k])
            m2s = jnp.max(m2, axis=0)
            m2v = jnp.full((LANES,), m2s, jnp.float32)
            cmin2 = big
            for k in range(nvr):
                cmin2 = jnp.minimum(cmin2, jnp.where(vs2[k] == m2v, gid[k], big))
            i2 = jnp.min(cmin2, axis=0)
            i2v = jnp.full((LANES,), i2, jnp.int32)
            sel = lane == j
            i1acc = jnp.where(sel, i1v, i1acc)
            i2acc = jnp.where(sel, i2v, i2acc)
        pos = (g * LANES + lane) * 2
        plsc.store_scatter(outc, [pos], i1acc)
        plsc.store_scatter(outc, [pos + 1], i2acc)
        return _

    lax.fori_loop(0, tpw // LANES, group_body, 0)
    pltpu.sync_copy(outc, out_hbm.at[pl.ds(base * 2, tpw * 2)])


def _sc_top2(aff, T, E):
    tpw = T // NWORKERS
    mesh = plsc.VectorSubcoreMesh(core_axis_name="c", subcore_axis_name="s")
    body = functools.partial(_sc_top2_body, tpw=tpw, n_experts=E)
    k = pl.kernel(
        body,
        mesh=mesh,
        out_type=jax.ShapeDtypeStruct((T * 2,), jnp.int32),
        scratch_types=[
            pltpu.VMEM((tpw * E,), jnp.float32),
            pltpu.VMEM((tpw * 2,), jnp.int32),
        ],
        compiler_params=pltpu.CompilerParams(needs_layout_passes=False),
    )
    return k(aff.reshape(T * E)).reshape(T, 2)


def kernel(hidden_states, W):
    S, B, H = hidden_states.shape
    E, _ = W.shape
    T = S * B
    x = hidden_states.reshape(T, H)
    logits, aff = _tc_proj_softmax(x, W, T, H, E)
    idx = _sc_top2(aff, T, E)
    return logits, aff, idx


# speedup vs baseline: 1.1926x; 1.1926x over previous
import functools

import jax
import jax.numpy as jnp
from jax.experimental import pallas as pl
from jax.experimental.pallas import tpu as pltpu

TOKEN_BLOCK = 2048


def _router_block_kernel(x_ref, w_ref, logits_ref, aff_ref, idx_ref, *, n_experts):
    x = x_ref[...]
    w = w_ref[...]
    logits = jax.lax.dot_general(
        x, w, (((1,), (1,)), ((), ())), preferred_element_type=jnp.float32
    )
    logits_ref[...] = logits

    m = jnp.max(logits, axis=1, keepdims=True)
    e = jnp.exp(logits - m)
    s = jnp.sum(e, axis=1, keepdims=True)
    aff = e / s
    aff_ref[...] = aff

    lane = jax.lax.broadcasted_iota(jnp.int32, aff.shape, 1)
    i1 = jnp.argmax(aff, axis=1, keepdims=True).astype(jnp.int32)
    masked = jnp.where(lane == i1, -jnp.inf, aff)
    i2 = jnp.argmax(masked, axis=1, keepdims=True).astype(jnp.int32)
    idx_ref[...] = jnp.concatenate([i1, i2], axis=1)


def kernel(hidden_states, W):
    S, B, H = hidden_states.shape
    E, _ = W.shape
    T = S * B
    x = hidden_states.reshape(T, H)
    tb = TOKEN_BLOCK
    grid = (T // tb,)

    logits, aff, idx = pl.pallas_call(
        functools.partial(_router_block_kernel, n_experts=E),
        grid=grid,
        in_specs=[
            pl.BlockSpec((tb, H), lambda i: (i, 0)),
            pl.BlockSpec((E, H), lambda i: (0, 0)),
        ],
        out_specs=[
            pl.BlockSpec((tb, E), lambda i: (i, 0)),
            pl.BlockSpec((tb, E), lambda i: (i, 0)),
            pl.BlockSpec((tb, 2), lambda i: (i, 0)),
        ],
        out_shape=[
            jax.ShapeDtypeStruct((T, E), jnp.float32),
            jax.ShapeDtypeStruct((T, E), jnp.float32),
            jax.ShapeDtypeStruct((T, 2), jnp.int32),
        ],
        compiler_params=pltpu.CompilerParams(
            dimension_semantics=("parallel",),
        ),
    )(x, W)
    return logits, aff, idx
